# baseline (speedup 1.0000x reference)
import functools

import jax
import jax.numpy as jnp
from jax import lax
from jax.experimental import pallas as pl
from jax.experimental.pallas import tpu as pltpu
from jax.experimental.pallas import tpu_sc as plsc

_info = plsc.get_sparse_core_info()
_NC, _NS, _L = _info.num_cores, _info.num_subcores, _info.num_lanes
_NW = _NC * _NS


def _proj_body(tab_ref, w_ref, b_ref, p0_ref, p1_ref):
    d = lax.dot_general(
        tab_ref[...], w_ref[...],
        dimension_numbers=(((1,), (1,)), ((), ())),
        preferred_element_type=jnp.float32,
    )
    p0_ref[...] = d[:, 0] + b_ref[0, 0]
    p1_ref[...] = d[:, 1] + b_ref[0, 1]


def _project_table(table, W, b2):
    V, D = table.shape
    BLK = 4096
    return pl.pallas_call(
        _proj_body,
        grid=((V + BLK - 1) // BLK,),
        in_specs=[
            pl.BlockSpec((BLK, D), lambda i: (i, 0)),
            pl.BlockSpec((2, D), lambda i: (0, 0)),
            pl.BlockSpec((1, 2), lambda i: (0, 0)),
        ],
        out_specs=[
            pl.BlockSpec((BLK,), lambda i: (i,)),
            pl.BlockSpec((BLK,), lambda i: (i,)),
        ],
        out_shape=[
            jax.ShapeDtypeStruct((V,), jnp.float32),
            jax.ShapeDtypeStruct((V,), jnp.float32),
        ],
    )(table, W, b2)


def _make_sc_gather(B):
    b_per_w = B // _NW
    mesh = plsc.VectorSubcoreMesh(core_axis_name="c", subcore_axis_name="s")

    @functools.partial(
        pl.kernel,
        mesh=mesh,
        out_type=jax.ShapeDtypeStruct((2 * B,), jnp.float32),
        scratch_types=[
            pltpu.VMEM((b_per_w,), jnp.int32),
            pltpu.VMEM((b_per_w,), jnp.float32),
            pltpu.VMEM((b_per_w,), jnp.float32),
            pltpu.SemaphoreType.DMA,
        ],
        compiler_params=pltpu.CompilerParams(use_tc_tiling_on_sc=False),
    )
    def gather_k(idx_hbm, p0_hbm, p1_hbm, out_hbm, idx_v, o0_v, o1_v, sem):
        wid = lax.axis_index("s") * _NC + lax.axis_index("c")
        base = wid * b_per_w
        pltpu.sync_copy(idx_hbm.at[pl.ds(base, b_per_w)], idx_v)
        pltpu.async_copy(p0_hbm.at[idx_v], o0_v, sem).wait()
        pltpu.async_copy(p1_hbm.at[idx_v], o1_v, sem).wait()
        pltpu.sync_copy(o0_v, out_hbm.at[pl.ds(base, b_per_w)])
        pltpu.sync_copy(o1_v, out_hbm.at[pl.ds(B + base, b_per_w)])

    return gather_k


def kernel(input_ids, table, W, b):
    B = input_ids.shape[0]
    idx = input_ids[:, 0].astype(jnp.int32)
    p0, p1 = _project_table(table, W, b.reshape(1, 2))
    out_flat = _make_sc_gather(B)(idx, p0, p1)
    return out_flat.reshape(2, B).T


# speedup vs baseline: 4.2746x; 4.2746x over previous
import functools

import jax
import jax.numpy as jnp
from jax import lax
from jax.experimental import pallas as pl
from jax.experimental.pallas import tpu as pltpu
from jax.experimental.pallas import tpu_sc as plsc

_info = plsc.get_sparse_core_info()
_NC, _NS, _L = _info.num_cores, _info.num_subcores, _info.num_lanes
_NW = _NC * _NS


def _proj_body(tabt_ref, w_ref, b_ref, p0_ref, p1_ref):
    d = lax.dot_general(
        w_ref[...], tabt_ref[...],
        dimension_numbers=(((1,), (0,)), ((), ())),
        preferred_element_type=jnp.float32,
    )
    p0_ref[...] = d[0, :] + b_ref[0, 0]
    p1_ref[...] = d[1, :] + b_ref[0, 1]


def _project_table(table_t, W, b2):
    D, V = table_t.shape
    BLK = 4096
    return pl.pallas_call(
        _proj_body,
        grid=((V + BLK - 1) // BLK,),
        in_specs=[
            pl.BlockSpec((D, BLK), lambda i: (0, i)),
            pl.BlockSpec((2, D), lambda i: (0, 0)),
            pl.BlockSpec((1, 2), lambda i: (0, 0)),
        ],
        out_specs=[
            pl.BlockSpec((BLK,), lambda i: (i,)),
            pl.BlockSpec((BLK,), lambda i: (i,)),
        ],
        out_shape=[
            jax.ShapeDtypeStruct((V,), jnp.float32),
            jax.ShapeDtypeStruct((V,), jnp.float32),
        ],
    )(table_t, W, b2)


def _make_sc_gather(B):
    b_per_w = B // _NW
    mesh = plsc.VectorSubcoreMesh(core_axis_name="c", subcore_axis_name="s")

    @functools.partial(
        pl.kernel,
        mesh=mesh,
        out_type=jax.ShapeDtypeStruct((2 * B,), jnp.float32),
        scratch_types=[
            pltpu.VMEM((b_per_w,), jnp.int32),
            pltpu.VMEM((b_per_w,), jnp.float32),
            pltpu.VMEM((b_per_w,), jnp.float32),
            pltpu.SemaphoreType.DMA,
        ],
        compiler_params=pltpu.CompilerParams(use_tc_tiling_on_sc=False),
    )
    def gather_k(idx_hbm, p0_hbm, p1_hbm, out_hbm, idx_v, o0_v, o1_v, sem):
        wid = lax.axis_index("s") * _NC + lax.axis_index("c")
        base = wid * b_per_w
        pltpu.sync_copy(idx_hbm.at[pl.ds(base, b_per_w)], idx_v)
        pltpu.async_copy(p0_hbm.at[idx_v], o0_v, sem).wait()
        pltpu.async_copy(p1_hbm.at[idx_v], o1_v, sem).wait()
        pltpu.sync_copy(o0_v, out_hbm.at[pl.ds(base, b_per_w)])
        pltpu.sync_copy(o1_v, out_hbm.at[pl.ds(B + base, b_per_w)])

    return gather_k


def kernel(input_ids, table, W, b):
    B = input_ids.shape[0]
    idx = input_ids[:, 0].astype(jnp.int32)
    p0, p1 = _project_table(table.T, W, b.reshape(1, 2))
    out_flat = _make_sc_gather(B)(idx, p0, p1)
    return out_flat.reshape(2, B).T


# speedup vs baseline: 5.5737x; 1.3039x over previous
import functools

import jax
import jax.numpy as jnp
from jax import lax
from jax.experimental import pallas as pl
from jax.experimental.pallas import tpu as pltpu
from jax.experimental.pallas import tpu_sc as plsc

_info = plsc.get_sparse_core_info()
_NC, _NS, _L = _info.num_cores, _info.num_subcores, _info.num_lanes
_NW = _NC * _NS


def _proj_body(tabt_ref, w_ref, b_ref, p0_ref, p1_ref):
    d = lax.dot_general(
        w_ref[...], tabt_ref[...],
        dimension_numbers=(((1,), (0,)), ((), ())),
        preferred_element_type=jnp.float32,
    )
    p0_ref[...] = d[0, :] + b_ref[0, 0]
    p1_ref[...] = d[1, :] + b_ref[0, 1]


def _project_table(table_t, W, b2):
    D, V = table_t.shape
    BLK = 16384
    return pl.pallas_call(
        _proj_body,
        grid=((V + BLK - 1) // BLK,),
        in_specs=[
            pl.BlockSpec((D, BLK), lambda i: (0, i)),
            pl.BlockSpec((2, D), lambda i: (0, 0)),
            pl.BlockSpec((1, 2), lambda i: (0, 0)),
        ],
        out_specs=[
            pl.BlockSpec((BLK,), lambda i: (i,)),
            pl.BlockSpec((BLK,), lambda i: (i,)),
        ],
        out_shape=[
            jax.ShapeDtypeStruct((V,), jnp.float32),
            jax.ShapeDtypeStruct((V,), jnp.float32),
        ],
    )(table_t, W, b2)


def _make_sc_gather(B):
    b_per_w = B // _NW
    mesh = plsc.VectorSubcoreMesh(core_axis_name="c", subcore_axis_name="s")

    @functools.partial(
        pl.kernel,
        mesh=mesh,
        out_type=jax.ShapeDtypeStruct((2 * B,), jnp.float32),
        scratch_types=[
            pltpu.VMEM((b_per_w,), jnp.int32),
            pltpu.VMEM((b_per_w,), jnp.float32),
            pltpu.VMEM((b_per_w,), jnp.float32),
            pltpu.SemaphoreType.DMA,
            pltpu.SemaphoreType.DMA,
        ],
        compiler_params=pltpu.CompilerParams(use_tc_tiling_on_sc=False),
    )
    def gather_k(idx_hbm, p0_hbm, p1_hbm, out_hbm, idx_v, o0_v, o1_v,
                 sem0, sem1):
        wid = lax.axis_index("s") * _NC + lax.axis_index("c")
        base = wid * b_per_w
        pltpu.sync_copy(idx_hbm.at[pl.ds(base, b_per_w)], idx_v)
        c0 = pltpu.async_copy(p0_hbm.at[idx_v], o0_v, sem0)
        c1 = pltpu.async_copy(p1_hbm.at[idx_v], o1_v, sem1)
        c0.wait()
        c1.wait()
        pltpu.sync_copy(o0_v, out_hbm.at[pl.ds(base, b_per_w)])
        pltpu.sync_copy(o1_v, out_hbm.at[pl.ds(B + base, b_per_w)])

    return gather_k


def kernel(input_ids, table, W, b):
    B = input_ids.shape[0]
    p0, p1 = _project_table(table.T, W, b.reshape(1, 2))
    out_flat = _make_sc_gather(B)(input_ids[:, 0].astype(jnp.int32), p0, p1)
    return out_flat.reshape(2, B).T


# speedup vs baseline: 5.9227x; 1.0626x over previous
import functools

import jax
import jax.numpy as jnp
from jax import lax
from jax.experimental import pallas as pl
from jax.experimental.pallas import tpu as pltpu
from jax.experimental.pallas import tpu_sc as plsc

_info = plsc.get_sparse_core_info()
_NC, _NS, _L = _info.num_cores, _info.num_subcores, _info.num_lanes
_NW = _NC * _NS


def _proj_body(tabt_ref, w_ref, b_ref, p0_ref, p1_ref):
    d = lax.dot_general(
        w_ref[...], tabt_ref[...],
        dimension_numbers=(((1,), (0,)), ((), ())),
        preferred_element_type=jnp.float32,
    )
    p0_ref[...] = d[0, :] + b_ref[0, 0]
    p1_ref[...] = d[1, :] + b_ref[0, 1]


def _project_table(table_t, W, b2):
    D, V = table_t.shape
    BLK = 32768
    return pl.pallas_call(
        _proj_body,
        grid=((V + BLK - 1) // BLK,),
        in_specs=[
            pl.BlockSpec((D, BLK), lambda i: (0, i)),
            pl.BlockSpec((2, D), lambda i: (0, 0)),
            pl.BlockSpec((1, 2), lambda i: (0, 0)),
        ],
        out_specs=[
            pl.BlockSpec((BLK,), lambda i: (i,)),
            pl.BlockSpec((BLK,), lambda i: (i,)),
        ],
        out_shape=[
            jax.ShapeDtypeStruct((V,), jnp.float32),
            jax.ShapeDtypeStruct((V,), jnp.float32),
        ],
    )(table_t, W, b2)


def _make_sc_gather(B):
    b_per_w = B // _NW
    mesh = plsc.VectorSubcoreMesh(core_axis_name="c", subcore_axis_name="s")

    @functools.partial(
        pl.kernel,
        mesh=mesh,
        out_type=jax.ShapeDtypeStruct((B // 128, 2, 128), jnp.float32),
        scratch_types=[
            pltpu.VMEM((b_per_w,), jnp.int32),
            pltpu.VMEM((b_per_w,), jnp.float32),
            pltpu.VMEM((b_per_w,), jnp.float32),
            pltpu.SemaphoreType.DMA,
            pltpu.SemaphoreType.DMA,
        ],
        compiler_params=pltpu.CompilerParams(use_tc_tiling_on_sc=False),
    )
    def gather_k(idx_hbm, p0_hbm, p1_hbm, out_hbm, idx_v, o0_v, o1_v,
                 sem0, sem1):
        wid = lax.axis_index("s") * _NC + lax.axis_index("c")
        base = wid * b_per_w
        pltpu.sync_copy(idx_hbm.at[pl.ds(base, b_per_w)], idx_v)
        c0 = pltpu.async_copy(p0_hbm.at[idx_v], o0_v, sem0)
        c1 = pltpu.async_copy(p1_hbm.at[idx_v], o1_v, sem1)
        c0.wait()
        c1.wait()
        for j in range(b_per_w // 128):
            blk = wid * (b_per_w // 128) + j
            pltpu.sync_copy(o0_v.at[pl.ds(j * 128, 128)], out_hbm.at[blk, 0])
            pltpu.sync_copy(o1_v.at[pl.ds(j * 128, 128)], out_hbm.at[blk, 1])

    return gather_k


def kernel(input_ids, table, W, b):
    B = input_ids.shape[0]
    p0, p1 = _project_table(table.T, W, b.reshape(1, 2))
    out3 = _make_sc_gather(B)(input_ids[:, 0].astype(jnp.int32), p0, p1)
    return out3.transpose(0, 2, 1).reshape(B, 2)


# speedup vs baseline: 6.1916x; 1.0454x over previous
import functools

import jax
import jax.numpy as jnp
from jax import lax
from jax.experimental import pallas as pl
from jax.experimental.pallas import tpu as pltpu
from jax.experimental.pallas import tpu_sc as plsc

_info = plsc.get_sparse_core_info()
_NC, _NS, _L = _info.num_cores, _info.num_subcores, _info.num_lanes
_NW = _NC * _NS


def _proj_body(tabt_ref, w_ref, b_ref, idst_ref, p0_ref, p1_ref, idx_ref):
    d = lax.dot_general(
        w_ref[...], tabt_ref[...],
        dimension_numbers=(((1,), (0,)), ((), ())),
        preferred_element_type=jnp.float32,
    )
    p0_ref[...] = d[0, :] + b_ref[0, 0]
    p1_ref[...] = d[1, :] + b_ref[0, 1]
    idx_ref[...] = idst_ref[0, :]


def _project_table(table_t, W, b2, ids_t):
    D, V = table_t.shape
    B = ids_t.shape[1]
    BLK = 32768
    return pl.pallas_call(
        _proj_body,
        grid=((V + BLK - 1) // BLK,),
        in_specs=[
            pl.BlockSpec((D, BLK), lambda i: (0, i)),
            pl.BlockSpec((2, D), lambda i: (0, 0)),
            pl.BlockSpec((1, 2), lambda i: (0, 0)),
            pl.BlockSpec((8, B), lambda i: (0, 0)),
        ],
        out_specs=[
            pl.BlockSpec((BLK,), lambda i: (i,)),
            pl.BlockSpec((BLK,), lambda i: (i,)),
            pl.BlockSpec((B,), lambda i: (0,)),
        ],
        out_shape=[
            jax.ShapeDtypeStruct((V,), jnp.float32),
            jax.ShapeDtypeStruct((V,), jnp.float32),
            jax.ShapeDtypeStruct((B,), jnp.int32),
        ],
    )(table_t, W, b2, ids_t)


def _make_sc_gather(B):
    b_per_w = B // _NW
    mesh = plsc.VectorSubcoreMesh(core_axis_name="c", subcore_axis_name="s")

    @functools.partial(
        pl.kernel,
        mesh=mesh,
        out_type=jax.ShapeDtypeStruct((B // 128, 2, 128), jnp.float32),
        scratch_types=[
            pltpu.VMEM((b_per_w,), jnp.int32),
            pltpu.VMEM((b_per_w,), jnp.float32),
            pltpu.VMEM((b_per_w,), jnp.float32),
            pltpu.SemaphoreType.DMA,
            pltpu.SemaphoreType.DMA,
        ],
        compiler_params=pltpu.CompilerParams(use_tc_tiling_on_sc=False),
    )
    def gather_k(idx_hbm, p0_hbm, p1_hbm, out_hbm, idx_v, o0_v, o1_v,
                 sem0, sem1):
        wid = lax.axis_index("s") * _NC + lax.axis_index("c")
        base = wid * b_per_w
        pltpu.sync_copy(idx_hbm.at[pl.ds(base, b_per_w)], idx_v)
        c0 = pltpu.async_copy(p0_hbm.at[idx_v], o0_v, sem0)
        c1 = pltpu.async_copy(p1_hbm.at[idx_v], o1_v, sem1)
        c0.wait()
        c1.wait()
        for j in range(b_per_w // 128):
            blk = wid * (b_per_w // 128) + j
            pltpu.sync_copy(o0_v.at[pl.ds(j * 128, 128)], out_hbm.at[blk, 0])
            pltpu.sync_copy(o1_v.at[pl.ds(j * 128, 128)], out_hbm.at[blk, 1])

    return gather_k


def kernel(input_ids, table, W, b):
    B = input_ids.shape[0]
    p0, p1, idx = _project_table(
        table.T, W, b.reshape(1, 2), input_ids.T.astype(jnp.int32)
    )
    out3 = _make_sc_gather(B)(idx, p0, p1)
    return out3.transpose(0, 2, 1).reshape(B, 2)


# speedup vs baseline: 6.2342x; 1.0069x over previous
import functools

import jax
import jax.numpy as jnp
from jax import lax
from jax.experimental import pallas as pl
from jax.experimental.pallas import tpu as pltpu
from jax.experimental.pallas import tpu_sc as plsc

_info = plsc.get_sparse_core_info()
_NC, _NS, _L = _info.num_cores, _info.num_subcores, _info.num_lanes
_NW = _NC * _NS


def _proj_body(tabt_ref, w_ref, b_ref, idst_ref, p0_ref, p1_ref, idx_ref):
    d = lax.dot_general(
        w_ref[...], tabt_ref[...],
        dimension_numbers=(((1,), (0,)), ((), ())),
        preferred_element_type=jnp.float32,
    )
    p0_ref[...] = d[0, :] + b_ref[0, 0]
    p1_ref[...] = d[1, :] + b_ref[0, 1]
    idx_ref[...] = idst_ref[0, :]


def _project_table(table_t, W, b2, ids_t):
    D, V = table_t.shape
    B = ids_t.shape[1]
    BLK = 50176
    return pl.pallas_call(
        _proj_body,
        grid=((V + BLK - 1) // BLK,),
        in_specs=[
            pl.BlockSpec((D, BLK), lambda i: (0, i)),
            pl.BlockSpec((2, D), lambda i: (0, 0)),
            pl.BlockSpec((1, 2), lambda i: (0, 0)),
            pl.BlockSpec((8, B), lambda i: (0, 0)),
        ],
        out_specs=[
            pl.BlockSpec((BLK,), lambda i: (i,)),
            pl.BlockSpec((BLK,), lambda i: (i,)),
            pl.BlockSpec((B,), lambda i: (0,)),
        ],
        out_shape=[
            jax.ShapeDtypeStruct((V,), jnp.float32),
            jax.ShapeDtypeStruct((V,), jnp.float32),
            jax.ShapeDtypeStruct((B,), jnp.int32),
        ],
    )(table_t, W, b2, ids_t)


def _make_sc_gather(B):
    b_per_w = B // _NW
    mesh = plsc.VectorSubcoreMesh(core_axis_name="c", subcore_axis_name="s")

    @functools.partial(
        pl.kernel,
        mesh=mesh,
        out_type=jax.ShapeDtypeStruct((B // 128, 2, 128), jnp.float32),
        scratch_types=[
            pltpu.VMEM((b_per_w,), jnp.int32),
            pltpu.VMEM((b_per_w,), jnp.float32),
            pltpu.VMEM((b_per_w,), jnp.float32),
            pltpu.SemaphoreType.DMA,
            pltpu.SemaphoreType.DMA,
        ],
        compiler_params=pltpu.CompilerParams(use_tc_tiling_on_sc=False),
    )
    def gather_k(idx_hbm, p0_hbm, p1_hbm, out_hbm, idx_v, o0_v, o1_v,
                 sem0, sem1):
        wid = lax.axis_index("s") * _NC + lax.axis_index("c")
        base = wid * b_per_w
        pltpu.sync_copy(idx_hbm.at[pl.ds(base, b_per_w)], idx_v)
        c0 = pltpu.async_copy(p0_hbm.at[idx_v], o0_v, sem0)
        c1 = pltpu.async_copy(p1_hbm.at[idx_v], o1_v, sem1)
        c0.wait()
        c1.wait()
        for j in range(b_per_w // 128):
            blk = wid * (b_per_w // 128) + j
            pltpu.sync_copy(o0_v.at[pl.ds(j * 128, 128)], out_hbm.at[blk, 0])
            pltpu.sync_copy(o1_v.at[pl.ds(j * 128, 128)], out_hbm.at[blk, 1])

    return gather_k


def kernel(input_ids, table, W, b):
    B = input_ids.shape[0]
    p0, p1, idx = _project_table(
        table.T, W, b.reshape(1, 2), input_ids.T.astype(jnp.int32)
    )
    out3 = _make_sc_gather(B)(idx, p0, p1)
    return out3.transpose(0, 2, 1).reshape(B, 2)
